# Initial kernel scaffold; baseline (speedup 1.0000x reference)
import jax
import jax.numpy as jnp
from jax.experimental import pallas as pl


def kernel(x, x_0, edge_index, weight1):
    raise NotImplementedError("write your pallas kernel here")


# speedup vs baseline: 42.2411x; 42.2411x over previous
import functools
import math

import jax
import jax.numpy as jnp
from jax import lax
from jax.experimental import pallas as pl
from jax.experimental.pallas import tpu as pltpu
from jax.experimental.pallas import tpu_sc as plsc

N = 10000
E = 320000
C = 128
ALPHA = 0.1
BETA = float(math.log(0.5 / 1 + 1))

NC = 2
NS = 16
NW = NC * NS

K = 80
EPT = E // NW
NCHUNK = EPT // K
EPC = E // NS
NCHUNK_D = EPC // K
NB = 3
NI = 6

RPT = 624
RLAST = 640
EBLK = 104
EBLK_L = 80


def _rsqrt_newton(v):
  s = v
  for _ in range(16):
    s = 0.5 * (s + v / s)
  return 1.0 / s


def _make_deg_scale_kernel():
  mesh = plsc.VectorSubcoreMesh(core_axis_name="c", subcore_axis_name="s")

  @functools.partial(
      pl.kernel,
      mesh=mesh,
      out_type=(
          jax.ShapeDtypeStruct((N,), jnp.float32),
          jax.ShapeDtypeStruct((N, C), jnp.float32),
      ),
      scratch_types=[
          pltpu.VMEM_SHARED((N,), jnp.float32),
          pltpu.VMEM((NCHUNK_D, K), jnp.int32),
          pltpu.VMEM((K,), jnp.float32),
          pltpu.VMEM((640,), jnp.float32),
          pltpu.VMEM((640,), jnp.float32),
          pltpu.VMEM((640,), jnp.float32),
          pltpu.VMEM((320, C), jnp.float32),
          pltpu.SemaphoreType.DMA,
      ],
      compiler_params=pltpu.CompilerParams(needs_layout_passes=False),
  )
  def deg_scale(dst3d_hbm, x_hbm, d_hbm, xs_hbm,
                cnt_shared, dstbuf, ones, zsrc, degloc, dloc, xbuf, dsem):
    c = lax.axis_index("c")
    s = lax.axis_index("s")
    w = c * NS + s

    def _fill(i, _):
      zsrc[pl.ds(i * 16, 16)] = jnp.zeros((16,), jnp.float32)
      return _
    lax.fori_loop(0, 40, _fill, None)
    for j in range(K // 16):
      ones[pl.ds(j * 16, 16)] = jnp.ones((16,), jnp.float32)

    @pl.when(s < NS - 1)
    def _():
      pltpu.sync_copy(zsrc.at[pl.ds(0, 640)], cnt_shared.at[pl.ds(s * 640, 640)])

    @pl.when(s == NS - 1)
    def _():
      pltpu.sync_copy(zsrc.at[pl.ds(0, 400)], cnt_shared.at[pl.ds(9600, 400)])

    pltpu.sync_copy(dst3d_hbm.at[s], dstbuf)
    plsc.subcore_barrier()

    group = 10
    for g in range(0, NCHUNK_D, group):
      descs = []
      for i in range(g, g + group):
        dsc = pltpu.make_async_copy(ones, cnt_shared.at[dstbuf.at[i]], dsem)
        dsc.start(add=True)
        descs.append(dsc)
      for dsc in descs:
        dsc.wait()
    plsc.subcore_barrier()

    def _epi(r0, nr):
      pltpu.sync_copy(cnt_shared.at[pl.ds(r0, nr)], degloc.at[pl.ds(0, nr)])

      def _dchunk(k, _):
        deg = degloc[pl.ds(k * 16, 16)] + 1.0
        dloc[pl.ds(k * 16, 16)] = _rsqrt_newton(deg)
        return _
      lax.fori_loop(0, nr // 16, _dchunk, None)

      pltpu.sync_copy(x_hbm.at[pl.ds(r0, nr), :], xbuf.at[pl.ds(0, nr), :])

      def _row(r, _):
        idx = jnp.broadcast_to(r, (16,)).astype(jnp.int32)
        dv = plsc.load_gather(dloc, [idx])
        for j in range(C // 16):
          sl = pl.ds(j * 16, 16)
          xbuf[r, sl] = xbuf[r, sl] * dv
        return _
      lax.fori_loop(0, nr, _row, None)

      pltpu.sync_copy(xbuf.at[pl.ds(0, nr), :], xs_hbm.at[pl.ds(r0, nr), :])
      pltpu.sync_copy(dloc.at[pl.ds(0, nr)], d_hbm.at[pl.ds(r0, nr)])

    @pl.when(w < NW - 1)
    def _():
      _epi(w * 320, 320)

    @pl.when(w == NW - 1)
    def _():
      _epi((NW - 1) * 320, 80)

  return deg_scale


def _make_agg_kernel():
  mesh = plsc.VectorSubcoreMesh(core_axis_name="c", subcore_axis_name="s")

  @functools.partial(
      pl.kernel,
      mesh=mesh,
      out_type=jax.ShapeDtypeStruct((NC, N, C), jnp.float32),
      scratch_types=[
          pltpu.VMEM_SHARED((N, C), jnp.float32),
          pltpu.VMEM((NI, K), jnp.int32),
          pltpu.VMEM((NI, K), jnp.int32),
          pltpu.VMEM((NB, K, C), jnp.float32),
          pltpu.VMEM((EBLK, C), jnp.float32),
          pltpu.VMEM((EBLK,), jnp.float32),
          pltpu.SemaphoreType.DMA((NI,)),
          pltpu.SemaphoreType.DMA((NB,)),
          pltpu.SemaphoreType.DMA((NB,)),
      ],
      compiler_params=pltpu.CompilerParams(needs_layout_passes=False),
  )
  def agg(xs_hbm, src_hbm, dst_hbm, d_hbm, hp_hbm,
          h_shared, sidx, didx, rows, tbuf, dbuf, isem, gsem, ssem):
    c = lax.axis_index("c")
    s = lax.axis_index("s")
    w = c * NS + s

    def _zero_tbuf(r, _):
      for j in range(C // 16):
        tbuf[r, pl.ds(j * 16, 16)] = jnp.zeros((16,), jnp.float32)
      return _

    def _init(r0, nrow, blk):
      @pl.when(c == 0)
      def _():
        pltpu.sync_copy(xs_hbm.at[pl.ds(r0, nrow), :],
                        h_shared.at[pl.ds(r0, nrow), :])

      @pl.when(c == 1)
      def _():
        for b in range(nrow // blk):
          pltpu.sync_copy(tbuf.at[pl.ds(0, blk), :],
                          h_shared.at[pl.ds(r0 + b * blk, blk), :])

    lax.fori_loop(0, EBLK, _zero_tbuf, None)

    @pl.when(s < NS - 1)
    def _():
      _init(s * RPT, RPT, EBLK)

    @pl.when(s == NS - 1)
    def _():
      _init((NS - 1) * RPT, RLAST, EBLK_L)

    plsc.subcore_barrier()

    def _idx_start(i):
      j = i % NI
      d1 = pltpu.make_async_copy(src_hbm.at[pl.ds(w * EPT + i * K, K)],
                                 sidx.at[j], isem.at[j])
      d2 = pltpu.make_async_copy(dst_hbm.at[pl.ds(w * EPT + i * K, K)],
                                 didx.at[j], isem.at[j])
      d1.start()
      d2.start()
      return (d1, d2)

    def _gather_start(i):
      b = i % NB
      dsc = pltpu.make_async_copy(xs_hbm.at[sidx.at[i % NI]], rows.at[b],
                                  gsem.at[b])
      dsc.start()
      return dsc

    def _scatter_start(i):
      b = i % NB
      dsc = pltpu.make_async_copy(rows.at[b], h_shared.at[didx.at[i % NI]],
                                  ssem.at[b])
      dsc.start(add=True)
      return dsc

    idxd = [None] * NI
    gd = [None] * NB
    sd = [None] * NB
    idx_waited = [False] * NCHUNK

    def _idx_wait(i):
      if not idx_waited[i]:
        for dsc in idxd[i % NI]:
          dsc.wait()
        idx_waited[i] = True

    for j in range(min(NI - 1, NCHUNK)):
      idxd[j] = _idx_start(j)
    for j in range(min(2, NCHUNK)):
      _idx_wait(j)
      gd[j] = _gather_start(j)

    for i in range(NCHUNK):
      nx = i + 2
      if nx < NCHUNK:
        b2 = nx % NB
        if nx >= NB:
          sd[b2].wait()
        _idx_wait(nx)
        gd[b2] = _gather_start(nx)
      b = i % NB
      gd[b].wait()
      sd[b] = _scatter_start(i)
      if i + NI - 1 < NCHUNK:
        idxd[(i + NI - 1) % NI] = _idx_start(i + NI - 1)
    for i in range(max(0, NCHUNK - NB), NCHUNK):
      sd[i % NB].wait()
    plsc.subcore_barrier()

    def _eblk(r0, nrow):
      pltpu.sync_copy(h_shared.at[pl.ds(r0, nrow), :],
                      tbuf.at[pl.ds(0, nrow), :])
      pltpu.sync_copy(d_hbm.at[pl.ds(r0, nrow)], dbuf.at[pl.ds(0, nrow)])

      def _row(r, _):
        idx = jnp.broadcast_to(r, (16,)).astype(jnp.int32)
        dv = plsc.load_gather(dbuf, [idx])
        for j in range(C // 16):
          sl = pl.ds(j * 16, 16)
          tbuf[r, sl] = tbuf[r, sl] * dv
        return _
      lax.fori_loop(0, nrow, _row, None)

      pltpu.sync_copy(tbuf.at[pl.ds(0, nrow), :],
                      hp_hbm.at[c, pl.ds(r0, nrow), :])

    @pl.when(s < NS - 1)
    def _():
      for blk in range(RPT // EBLK):
        _eblk(s * RPT + blk * EBLK, EBLK)

    @pl.when(s == NS - 1)
    def _():
      for blk in range(RLAST // EBLK_L):
        _eblk((NS - 1) * RPT + blk * EBLK_L, EBLK_L)

  return agg


def _update_body(hp_ref, x0_ref, w1_ref, out_ref):
  m = (1.0 - ALPHA) * (hp_ref[0] + hp_ref[1]) + ALPHA * x0_ref[...]
  row = lax.broadcasted_iota(jnp.int32, (C, C), 0)
  col = lax.broadcasted_iota(jnp.int32, (C, C), 1)
  eye = jnp.where(row == col, 1.0, 0.0).astype(jnp.float32)
  wmod = (1.0 - BETA) * eye + BETA * w1_ref[...]
  out_ref[...] = jnp.dot(m, wmod, preferred_element_type=jnp.float32)


def kernel(x, x_0, edge_index, weight1):
  src = edge_index[0].astype(jnp.int32)
  dst = edge_index[1].astype(jnp.int32)
  dst3d_s = dst.reshape(NS, NCHUNK_D, K)

  d, xs = _make_deg_scale_kernel()(dst3d_s, x)
  hp = _make_agg_kernel()(xs, src, dst, d)

  rblk = 1000
  out = pl.pallas_call(
      _update_body,
      grid=(N // rblk,),
      in_specs=[
          pl.BlockSpec((NC, rblk, C), lambda i: (0, i, 0)),
          pl.BlockSpec((rblk, C), lambda i: (i, 0)),
          pl.BlockSpec((C, C), lambda i: (0, 0)),
      ],
      out_specs=pl.BlockSpec((rblk, C), lambda i: (i, 0)),
      out_shape=jax.ShapeDtypeStruct((N, C), jnp.float32),
  )(hp, x_0, weight1)
  return out
